# baseline (speedup 1.0000x reference)
import dataclasses
import functools

import jax
import jax.numpy as jnp
from jax import lax
from jax.experimental import pallas as pl
from jax.experimental.pallas import tpu as pltpu
from jax.experimental.pallas import tpu_sc as plsc

N = 10000
E = 320000
REL = 500
EH = 128

NW = 32
EPW = E // NW
CH = 80
NCH = EPW // CH


def _prep_body(rel_ref, ww_ref, wrel_ref, dst_ref, et_ref,
               r2_ref, ro_ref, pi_ref):
    r = rel_ref[...]
    dn = (((1,), (0,)), ((), ()))
    r2 = lax.dot_general(r, ww_ref[...], dn,
                         precision=lax.Precision.HIGHEST,
                         preferred_element_type=jnp.float32)
    r2_ref[...] = r2
    ro_ref[...] = lax.dot_general(r, wrel_ref[...], dn,
                                  precision=lax.Precision.HIGHEST,
                                  preferred_element_type=jnp.float32)
    pi_ref[...] = dst_ref[...] * REL + et_ref[...]


_prep = pl.pallas_call(
    _prep_body,
    out_shape=[jax.ShapeDtypeStruct((REL, EH), jnp.float32),
               jax.ShapeDtypeStruct((REL, EH), jnp.float32),
               jax.ShapeDtypeStruct((NW, NCH, CH), jnp.int32)],
)


def _pmat_body(x_ref, r2_ref, p_ref):
    dnt = (((1,), (1,)), ((), ()))
    p_ref[...] = lax.dot_general(x_ref[...], r2_ref[...], dnt,
                                 precision=lax.Precision.HIGHEST,
                                 preferred_element_type=jnp.float32)


_pmat = pl.pallas_call(
    _pmat_body,
    grid=(5,),
    in_specs=[pl.BlockSpec((N // 5, EH), lambda i: (i, 0)),
              pl.BlockSpec((REL, EH), lambda i: (0, 0))],
    out_specs=pl.BlockSpec((N // 5, REL), lambda i: (i, 0)),
    out_shape=jax.ShapeDtypeStruct((N, REL), jnp.float32),
)


def _softmax_body(dp_ref, w_ref):
    d = dp_ref[...]
    m = jnp.max(d)
    e = jnp.exp(d - m)
    w_ref[...] = e / jnp.sum(e)


_softmax = pl.pallas_call(
    _softmax_body,
    out_shape=jax.ShapeDtypeStruct((NW, NCH, CH), jnp.float32),
)


def _final_body(p_ref, out_ref):
    out_ref[...] = jnp.maximum(p_ref[0] + p_ref[1], 0.0)


_final = pl.pallas_call(
    _final_body,
    out_shape=jax.ShapeDtypeStruct((N, EH), jnp.float32),
)


@functools.cache
def _sc_kernels():
    mesh = plsc.VectorSubcoreMesh(core_axis_name="c", subcore_axis_name="s")
    cp = pltpu.CompilerParams()
    if "needs_layout_passes" in pltpu.CompilerParams.__dataclass_fields__:
        cp = dataclasses.replace(cp, needs_layout_passes=False)

    @functools.partial(
        pl.kernel,
        out_type=jax.ShapeDtypeStruct((NW, NCH, CH), jnp.float32),
        mesh=mesh,
        compiler_params=cp,
        scratch_types=[
            pltpu.VMEM((NCH, CH), jnp.int32),
            pltpu.VMEM((NCH, CH), jnp.int32),
            pltpu.VMEM((NCH, CH), jnp.int32),
            pltpu.VMEM((CH, EH), jnp.float32),
            pltpu.VMEM((CH, EH), jnp.float32),
            pltpu.VMEM((CH,), jnp.float32),
            pltpu.VMEM((CH, EH), jnp.float32),
            pltpu.VMEM((CH, EH), jnp.float32),
            pltpu.VMEM((CH,), jnp.float32),
            pltpu.VMEM((16, 17), jnp.float32),
            pltpu.VMEM((NCH, CH), jnp.float32),
            pltpu.SemaphoreType.DMA,
            pltpu.SemaphoreType.DMA,
            pltpu.SemaphoreType.DMA,
            pltpu.SemaphoreType.DMA,
            pltpu.SemaphoreType.DMA,
            pltpu.SemaphoreType.DMA,
        ],
    )
    def _pass1(x_hbm, pf_hbm, src_hbm, dst_hbm, pi_hbm, dp_hbm,
               src_v, dst_v, pi_v, xsa_v, xda_v, pva_v, xsb_v, xdb_v, pvb_v,
               tb_v, dp_v, sa0, sa1, sa2, sb0, sb1, sb2):
        cid = lax.axis_index("c")
        sid = lax.axis_index("s")
        wid = cid * 16 + sid
        pltpu.sync_copy(src_hbm.at[wid], src_v)
        pltpu.sync_copy(dst_hbm.at[wid], dst_v)
        pltpu.sync_copy(pi_hbm.at[wid], pi_v)
        lanes = lax.iota(jnp.int32, 16)

        def fire(k, xs_v, xd_v, pv_v, s0, s1, s2):
            c1 = pltpu.async_copy(x_hbm.at[src_v.at[k]], xs_v, s0)
            c2 = pltpu.async_copy(x_hbm.at[dst_v.at[k]], xd_v, s1)
            c3 = pltpu.async_copy(pf_hbm.at[pi_v.at[k]], pv_v, s2)
            return (c1, c2, c3)

        def compute(k, xs_v, xd_v, pv_v):
            @pl.loop(0, CH // 16)
            def _grp(g):
                e0 = g * 16
                for j in range(16):
                    b = e0 + j
                    acc = xs_v[b, pl.ds(0, 16)] * xd_v[b, pl.ds(0, 16)]
                    for c in range(1, 8):
                        sl = pl.ds(c * 16, 16)
                        acc = acc + xs_v[b, sl] * xd_v[b, sl]
                    tb_v[j, pl.ds(0, 16)] = acc
                dpv = pv_v[pl.ds(e0, 16)]
                for j2 in range(16):
                    jv = jnp.full((16,), j2, jnp.int32)
                    dpv = dpv + plsc.load_gather(tb_v, [lanes, jv])
                dp_v[k, pl.ds(e0, 16)] = dpv

        ca = fire(0, xsa_v, xda_v, pva_v, sa0, sa1, sa2)
        cb = fire(1, xsb_v, xdb_v, pvb_v, sb0, sb1, sb2)
        del ca, cb

        @pl.loop(0, NCH // 2)
        def _pair(i):
            k0 = i * 2
            k1 = k0 + 1
            pltpu.make_async_copy(x_hbm.at[src_v.at[k0]], xsa_v, sa0).wait()
            pltpu.make_async_copy(x_hbm.at[dst_v.at[k0]], xda_v, sa1).wait()
            pltpu.make_async_copy(pf_hbm.at[pi_v.at[k0]], pva_v, sa2).wait()
            compute(k0, xsa_v, xda_v, pva_v)
            fire(k0 + 2, xsa_v, xda_v, pva_v, sa0, sa1, sa2)
            pltpu.make_async_copy(x_hbm.at[src_v.at[k1]], xsb_v, sb0).wait()
            pltpu.make_async_copy(x_hbm.at[dst_v.at[k1]], xdb_v, sb1).wait()
            pltpu.make_async_copy(pf_hbm.at[pi_v.at[k1]], pvb_v, sb2).wait()
            compute(k1, xsb_v, xdb_v, pvb_v)

            @pl.when(k1 + 2 < NCH)
            def _fb():
                fire(k1 + 2, xsb_v, xdb_v, pvb_v, sb0, sb1, sb2)

        pltpu.make_async_copy(x_hbm.at[src_v.at[NCH - 1]], xsa_v, sa0).wait()
        pltpu.make_async_copy(x_hbm.at[dst_v.at[NCH - 1]], xda_v, sa1).wait()
        pltpu.make_async_copy(pf_hbm.at[pi_v.at[NCH - 1]], pva_v, sa2).wait()
        compute(NCH - 1, xsa_v, xda_v, pva_v)

        pltpu.sync_copy(dp_v, dp_hbm.at[wid])

    @functools.partial(
        pl.kernel,
        out_type=jax.ShapeDtypeStruct((2, N, EH), jnp.float32),
        mesh=mesh,
        compiler_params=cp,
        scratch_types=[
            pltpu.VMEM((4, CH), jnp.int32),
            pltpu.VMEM((CH, EH), jnp.float32),
            pltpu.VMEM((CH, EH), jnp.float32),
            pltpu.VMEM((4, CH), jnp.int32),
            pltpu.VMEM((CH, EH), jnp.float32),
            pltpu.VMEM((CH, EH), jnp.float32),
            pltpu.VMEM_SHARED((N, EH), jnp.float32),
            pltpu.SemaphoreType.DMA,
            pltpu.SemaphoreType.DMA,
            pltpu.SemaphoreType.DMA,
            pltpu.SemaphoreType.DMA,
        ],
    )
    def _pass2(x_hbm, r2_hbm, pk_hbm, out_hbm,
               pka_v, xsa_v, rra_v, pkb_v, xsb_v, rrb_v, acc_sh,
               sa0, sa1, sb0, sb1):
        cid = lax.axis_index("c")
        sid = lax.axis_index("s")
        wid = cid * 16 + sid
        zero16 = jnp.zeros((16,), jnp.float32)

        @pl.loop(0, CH)
        def _zrow(i):
            for c in range(8):
                xsa_v[i, pl.ds(c * 16, 16)] = zero16

        row0 = pl.multiple_of(sid * 624, 16)
        for q in range(7):
            pltpu.sync_copy(xsa_v, acc_sh.at[pl.ds(row0 + q * CH, CH)])
        pltpu.sync_copy(xsa_v.at[pl.ds(0, 64)],
                        acc_sh.at[pl.ds(row0 + 560, 64)])

        @pl.when(sid == 15)
        def _ztail():
            pltpu.sync_copy(xsa_v.at[pl.ds(0, 16)],
                            acc_sh.at[pl.ds(9984, 16)])

        plsc.subcore_barrier()

        def fire(k, pk_v, xs_v, rr_v, s0, s1):
            pltpu.sync_copy(pk_hbm.at[wid, k], pk_v)
            pltpu.async_copy(x_hbm.at[pk_v.at[0]], xs_v, s0)
            pltpu.async_copy(r2_hbm.at[pk_v.at[2]], rr_v, s1)

        def wait(pk_v, xs_v, rr_v, s0, s1):
            pltpu.make_async_copy(x_hbm.at[pk_v.at[0]], xs_v, s0).wait()
            pltpu.make_async_copy(r2_hbm.at[pk_v.at[2]], rr_v, s1).wait()

        def compute_scatter(pk_v, xs_v, rr_v):
            @pl.loop(0, CH // 16)
            def _grp(g):
                e0 = g * 16
                wg = plsc.bitcast(pk_v[3, pl.ds(e0, 16)], jnp.float32)
                for j in range(16):
                    b = e0 + j
                    wvec = jnp.full((16,), wg[j], jnp.float32)
                    for c in range(8):
                        sl = pl.ds(c * 16, 16)
                        xs_v[b, sl] = (xs_v[b, sl] + rr_v[b, sl]) * wvec

            pltpu.sync_copy(xs_v, acc_sh.at[pk_v.at[1]], add=True)

        fire(0, pka_v, xsa_v, rra_v, sa0, sa1)
        fire(1, pkb_v, xsb_v, rrb_v, sb0, sb1)

        @pl.loop(0, NCH // 2)
        def _pair(i):
            k0 = i * 2
            k1 = k0 + 1
            wait(pka_v, xsa_v, rra_v, sa0, sa1)
            compute_scatter(pka_v, xsa_v, rra_v)
            fire(k0 + 2, pka_v, xsa_v, rra_v, sa0, sa1)
            wait(pkb_v, xsb_v, rrb_v, sb0, sb1)
            compute_scatter(pkb_v, xsb_v, rrb_v)

            @pl.when(k1 + 2 < NCH)
            def _fb():
                fire(k1 + 2, pkb_v, xsb_v, rrb_v, sb0, sb1)

        wait(pka_v, xsa_v, rra_v, sa0, sa1)
        compute_scatter(pka_v, xsa_v, rra_v)

        plsc.subcore_barrier()
        pltpu.sync_copy(acc_sh.at[pl.ds(row0, 624)],
                        out_hbm.at[cid, pl.ds(row0, 624)])

        @pl.when(sid == 15)
        def _wtail():
            pltpu.sync_copy(acc_sh.at[pl.ds(9984, 16)],
                            out_hbm.at[cid, pl.ds(9984, 16)])

    return _pass1, _pass2


def kernel(x, edge_index, edge_type, rel_emb, res_att, W_ww, W_rel):
    pass1, pass2 = _sc_kernels()
    src = edge_index[0].reshape(NW, NCH, CH)
    dst = edge_index[1].reshape(NW, NCH, CH)
    et = edge_type.reshape(NW, NCH, CH)
    r2, rel_out, pidx = _prep(rel_emb, W_ww, W_rel, dst, et)
    P = _pmat(x, r2)
    dp = pass1(x, P.reshape(N * REL), src, dst, pidx)
    w = _softmax(dp)
    pk2 = jnp.stack(
        [src, dst, et, lax.bitcast_convert_type(w, jnp.int32)], axis=2)
    partials = pass2(x, r2, pk2)
    out = _final(partials)
    return (out, rel_out, res_att)
